# baseline (speedup 1.0000x reference)
import jax
import jax.numpy as jnp
from jax import lax
from jax.experimental import pallas as pl
from jax.experimental.pallas import tpu as pltpu
from jax.experimental.pallas import tpu_sc as plsc

_LANES = 16
_D = 64
_B = 16384
_NW = 32
_B_PER_W = _B // _NW
_V = 1000000
_VT = 7813
_OWN_COLS = 32768
_OWN_TILES = _OWN_COLS // 128
_WIN_T = 4
_WIN_COLS = _WIN_T * 128
_SROWS = _B + _NW
_SW = 128


def _sweep_body(users_hbm, items_hbm, utT_hbm, itT_hbm, ustage, istage,
                scan_v, bin_v, win_v, row_v, ridx_v, swp_sem, scat_sem):
    wid = lax.axis_index("s") * 2 + lax.axis_index("c")
    col0 = wid * _OWN_COLS
    tile0 = wid * _OWN_TILES
    ntiles = jnp.maximum(jnp.minimum(_OWN_TILES, _VT - tile0), 0)
    nwin = (ntiles + _WIN_T - 1) // _WIN_T
    dump_row = _B + wid
    lane = lax.iota(jnp.int32, _LANES)

    for tbl_hbm, idx_hbm, stage in ((utT_hbm, users_hbm, ustage),
                                    (itT_hbm, items_hbm, istage)):
        pltpu.sync_copy(idx_hbm, scan_v)

        def scan_step(g, nbin):
            v = scan_v[pl.ds(g * _LANES, _LANES)]
            m = (v >> 15) == wid
            packed = (v & 32767) | ((g * _LANES + lane) << 15)
            pc = plsc.cumsum(jnp.where(m, 1, 0))
            plsc.store_scatter(bin_v, [nbin + pc - 1], packed, mask=m)
            return nbin + pc[_LANES - 1]

        nbin = lax.fori_loop(0, _B // _LANES, scan_step, jnp.int32(0))
        nbv = (nbin + _LANES - 1) // _LANES

        def window(b, _):
            st_rel = jnp.minimum(b * _WIN_T, ntiles - _WIN_T)
            st_abs = tile0 + st_rel
            copies = []
            for ct in range(_D // 8):
                for t in range(_WIN_T):
                    src_col = pl.multiple_of((st_abs + t) * 128, 128)
                    copies.append(pltpu.async_copy(
                        tbl_hbm.at[pl.ds(ct * 8, 8), pl.ds(src_col, 128)],
                        win_v.at[pl.ds(ct * 8, 8), pl.ds(t * 128, 128)],
                        swp_sem))
            for cp in copies:
                cp.wait()

            def bin_step(k, _):
                p = bin_v[pl.ds(k * _LANES, _LANES)]
                valid = (k * _LANES + lane) < nbin
                local = p & 32767
                m = valid & ((local >> 9) == b)

                @pl.when(jnp.any(m))
                def _process():
                    r = jnp.where(m, p >> 15, dump_row)
                    i_rel = jnp.where(m, local - st_rel * 128, 0)
                    for c in range(_D):
                        cc = jnp.full((_LANES,), c, jnp.int32)
                        val = plsc.load_gather(win_v, [cc, i_rel])
                        plsc.store_scatter(row_v, [lane, cc], val)
                    ridx_v[...] = r
                    pltpu.async_copy(row_v, stage.at[ridx_v],
                                     scat_sem).wait()

                return _

            lax.fori_loop(0, nbv, bin_step, None)
            return _

        @pl.when(nwin > 0)
        def _sweep():
            lax.fori_loop(0, nwin, window, None)


def _dot_body(ustage, istage, out_hbm, urows_v, irows_v, out_v, usem, isem):
    wid = lax.axis_index("s") * 2 + lax.axis_index("c")
    base = wid * _B_PER_W
    lane = lax.iota(jnp.int32, _LANES)
    half = _B_PER_W // 2

    for h in range(2):
        r0 = base + h * half
        cu = pltpu.async_copy(ustage.at[pl.ds(r0, half)], urows_v, usem)
        ci = pltpu.async_copy(istage.at[pl.ds(r0, half)], irows_v, isem)
        cu.wait()
        ci.wait()

        def group(g, _):
            g0 = g * _LANES
            sums = jnp.zeros((_LANES,), jnp.float32)
            for k in range(_LANES):
                acc = jnp.zeros((_LANES,), jnp.float32)
                for c in range(_D // _LANES):
                    csl = pl.ds(c * _LANES, _LANES)
                    acc = acc + urows_v[g0 + k, csl] * irows_v[g0 + k, csl]
                sums = jnp.where(lane == k, jnp.sum(acc), sums)
            out_v[pl.ds(h * half + g0, _LANES)] = 1.0 / (1.0 + jnp.exp(-sums))
            return _

        lax.fori_loop(0, half // _LANES, group, None)

    pltpu.sync_copy(out_v, out_hbm.at[pl.ds(base, _B_PER_W)])


@jax.jit
def _pure_mf_sc(users, items, utT, itT):
    mesh = plsc.VectorSubcoreMesh(core_axis_name="c", subcore_axis_name="s")
    params = pltpu.CompilerParams(needs_layout_passes=False)
    ustage, istage = pl.kernel(
        _sweep_body,
        mesh=mesh,
        compiler_params=params,
        out_type=(jax.ShapeDtypeStruct((_SROWS, _SW), jnp.float32),
                  jax.ShapeDtypeStruct((_SROWS, _SW), jnp.float32)),
        scratch_types=[
            pltpu.VMEM((_B,), jnp.int32),
            pltpu.VMEM((_B,), jnp.int32),
            pltpu.VMEM((_D, _WIN_COLS), jnp.float32),
            pltpu.VMEM((_LANES, _SW), jnp.float32),
            pltpu.VMEM((_LANES,), jnp.int32),
            pltpu.SemaphoreType.DMA,
            pltpu.SemaphoreType.DMA,
        ],
    )(users, items, utT, itT)

    return pl.kernel(
        _dot_body,
        mesh=mesh,
        compiler_params=params,
        out_type=jax.ShapeDtypeStruct((_B,), jnp.float32),
        scratch_types=[
            pltpu.VMEM((_B_PER_W // 2, _SW), jnp.float32),
            pltpu.VMEM((_B_PER_W // 2, _SW), jnp.float32),
            pltpu.VMEM((_B_PER_W,), jnp.float32),
            pltpu.SemaphoreType.DMA,
            pltpu.SemaphoreType.DMA,
        ],
    )(ustage, istage)


def kernel(users, items, user_table, item_table):
    return _pure_mf_sc(users, items, user_table.T, item_table.T)


# speedup vs baseline: 3.5348x; 3.5348x over previous
import jax
import jax.numpy as jnp
from jax import lax
from jax.experimental import pallas as pl
from jax.experimental.pallas import tpu as pltpu
from jax.experimental.pallas import tpu_sc as plsc

_LANES = 16
_D = 64
_B = 16384
_NW = 32
_B_PER_W = _B // _NW
_V = 1000000
_VT = 7813
_OWN_COLS = 32768
_OWN_TILES = _OWN_COLS // 128
_WIN_T = 4
_WIN_COLS = _WIN_T * 128
_SROWS = _B + _NW
_SW = 128
_BATCH = 64
_SEGV = 128
_PEND = _SEGV * _LANES


def _sweep_body(users_hbm, items_hbm, utT_hbm, itT_hbm, ustage, istage,
                scan_v, bin_v, win_v, win2_v, pend_v,
                batch_v, bidx_v, swp_sem, scat_sem):
    wid = lax.axis_index("s") * 2 + lax.axis_index("c")
    col0 = wid * _OWN_COLS
    tile0 = wid * _OWN_TILES
    ntiles = jnp.maximum(jnp.minimum(_OWN_TILES, _VT - tile0), 0)
    nwin = (ntiles + _WIN_T - 1) // _WIN_T
    dump_row = _B + wid
    lane = lax.iota(jnp.int32, _LANES)

    for tbl_hbm, idx_hbm, stage in ((utT_hbm, users_hbm, ustage),
                                    (itT_hbm, items_hbm, istage)):
        pltpu.sync_copy(idx_hbm, scan_v)

        def scan_step(g, nbin):
            v = scan_v[pl.ds(g * _LANES, _LANES)]
            m = (v >> 15) == wid
            packed = (v & 32767) | ((g * _LANES + lane) << 15)
            plsc.store_compressed(bin_v.at[pl.ds(nbin, _LANES)], packed, mask=m)
            return nbin + plsc.all_reduce_population_count(m)[0]

        nbin = lax.fori_loop(0, _B // _LANES, scan_step, jnp.int32(0))
        nbv = (nbin + _LANES - 1) // _LANES

        def issue(b, buf):
            st_abs = tile0 + jnp.minimum(b * _WIN_T, ntiles - _WIN_T)
            for ct in range(_D // 8):
                for t in range(_WIN_T):
                    src_col = pl.multiple_of((st_abs + t) * 128, 128)
                    pltpu.async_copy(
                        tbl_hbm.at[pl.ds(ct * 8, 8), pl.ds(src_col, 128)],
                        buf.at[pl.ds(ct * 32 + t * 8, 8)], swp_sem)

        def flush_group(p, m, st_rel, buf, nrow, cnt):
            slot = cnt & 1
            r = jnp.where(m, p >> 15, dump_row)
            i_rel = jnp.where(m, (p & 32767) - st_rel * 128, 0)
            rowt = nrow + lane
            trow = (i_rel >> 7) << 3
            il = i_rel & 127
            for c in range(_D):
                wrow = trow + ((c >> 3) * (8 * _WIN_T) + (c & 7))
                val = plsc.load_gather(buf, [wrow, il])
                cc = jnp.full((_LANES,), c, jnp.int32)
                plsc.store_scatter(batch_v.at[slot], [rowt, cc], val)
            bidx_v[slot, pl.ds(nrow, _LANES)] = r
            nrow = nrow + _LANES
            full = nrow == _BATCH

            @pl.when(full & (cnt >= 1))
            def _drain_one():
                pltpu.make_async_copy(batch_v.at[0], stage.at[bidx_v.at[0]],
                                      scat_sem).wait()

            @pl.when(full)
            def _flush():
                pltpu.async_copy(batch_v.at[slot], stage.at[bidx_v.at[slot]],
                                 scat_sem)
                nslot = (cnt + 1) & 1
                for q in range(_BATCH // _LANES):
                    bidx_v[nslot, pl.ds(q * _LANES, _LANES)] = (
                        jnp.full((_LANES,), dump_row, jnp.int32))

            cnt = jnp.where(full, cnt + 1, cnt)
            nrow = jnp.where(full, 0, nrow)
            return nrow, cnt

        def drain_extract(b, buf, nrow, cnt):
            pltpu.make_async_copy(
                stage.at[pl.ds(0, 8 * _D // 8 * _WIN_T)], buf,
                swp_sem).wait()
            st_rel = jnp.minimum(b * _WIN_T, ntiles - _WIN_T)

            nseg = (nbv + _SEGV - 1) // _SEGV

            def segment(si, carry):
                nrow, cnt = carry
                k0 = si * _SEGV
                kn = jnp.minimum(nbv - k0, _SEGV)

                def scan1(k, npend):
                    kk = k0 + k
                    p = bin_v[pl.ds(kk * _LANES, _LANES)]
                    valid = (kk * _LANES + lane) < nbin
                    m = valid & (((p & 32767) >> 9) == b)
                    plsc.store_compressed(pend_v.at[pl.ds(npend, _LANES)],
                                          p, mask=m)
                    return npend + plsc.all_reduce_population_count(m)[0]

                npend = lax.fori_loop(0, kn, scan1, jnp.int32(0))

                def flush1(q, carry):
                    nrow, cnt = carry
                    pp = pend_v[pl.ds(q * _LANES, _LANES)]
                    mm = (q * _LANES + lane) < npend
                    return flush_group(pp, mm, st_rel, buf, nrow, cnt)

                nfl = (npend + _LANES - 1) // _LANES
                return lax.fori_loop(0, nfl, flush1, (nrow, cnt))

            return lax.fori_loop(0, nseg, segment, (nrow, cnt))

        def window_pair(bb, carry):
            nrow, cnt = carry
            b0 = bb * 2
            issue(b0 + 1, win2_v)
            nrow, cnt = drain_extract(b0, win_v, nrow, cnt)

            @pl.when(b0 + 2 < nwin)
            def _prefetch():
                issue(b0 + 2, win_v)

            nrow, cnt = drain_extract(b0 + 1, win2_v, nrow, cnt)
            return nrow, cnt

        @pl.when(nwin > 0)
        def _sweep():
            for q in range(_BATCH // _LANES):
                bidx_v[0, pl.ds(q * _LANES, _LANES)] = (
                    jnp.full((_LANES,), dump_row, jnp.int32))
            issue(0, win_v)
            nrow, cnt = lax.fori_loop(0, nwin // 2, window_pair,
                                      (jnp.int32(0), jnp.int32(0)))

            @pl.when(nrow > 0)
            def _last():
                pltpu.async_copy(batch_v.at[cnt & 1],
                                 stage.at[bidx_v.at[cnt & 1]], scat_sem)

            ndrain = jnp.minimum(cnt, 1) + jnp.where(nrow > 0, 1, 0)

            def drain_step(q, _):
                pltpu.make_async_copy(batch_v.at[0], stage.at[bidx_v.at[0]],
                                      scat_sem).wait()
                return _

            lax.fori_loop(0, ndrain, drain_step, None)


def _dot_body(ustage, istage, out_hbm, urows_v, irows_v, out_v, usem, isem):
    wid = lax.axis_index("s") * 2 + lax.axis_index("c")
    base = wid * _B_PER_W
    lane = lax.iota(jnp.int32, _LANES)
    half = _B_PER_W // 2

    for h in range(2):
        r0 = base + h * half
        cu = pltpu.async_copy(ustage.at[pl.ds(r0, half)], urows_v, usem)
        ci = pltpu.async_copy(istage.at[pl.ds(r0, half)], irows_v, isem)
        cu.wait()
        ci.wait()

        def group(g, _):
            g0 = g * _LANES
            sums = jnp.zeros((_LANES,), jnp.float32)
            for k in range(_LANES):
                acc = jnp.zeros((_LANES,), jnp.float32)
                for c in range(_D // _LANES):
                    csl = pl.ds(c * _LANES, _LANES)
                    acc = acc + urows_v[g0 + k, csl] * irows_v[g0 + k, csl]
                sums = jnp.where(lane == k, jnp.sum(acc), sums)
            out_v[pl.ds(h * half + g0, _LANES)] = 1.0 / (1.0 + jnp.exp(-sums))
            return _

        lax.fori_loop(0, half // _LANES, group, None)

    pltpu.sync_copy(out_v, out_hbm.at[pl.ds(base, _B_PER_W)])


@jax.jit
def _pure_mf_sc(users, items, utT, itT):
    mesh = plsc.VectorSubcoreMesh(core_axis_name="c", subcore_axis_name="s")
    params = pltpu.CompilerParams(needs_layout_passes=False)
    ustage, istage = pl.kernel(
        _sweep_body,
        mesh=mesh,
        compiler_params=params,
        out_type=(jax.ShapeDtypeStruct((_SROWS, _SW), jnp.float32),
                  jax.ShapeDtypeStruct((_SROWS, _SW), jnp.float32)),
        scratch_types=[
            pltpu.VMEM((_B,), jnp.int32),
            pltpu.VMEM((_B,), jnp.int32),
            pltpu.VMEM((_D * _WIN_T, 128), jnp.float32),
            pltpu.VMEM((_D * _WIN_T, 128), jnp.float32),
            pltpu.VMEM((_PEND + _LANES,), jnp.int32),
            pltpu.VMEM((2, _BATCH, _SW), jnp.float32),
            pltpu.VMEM((2, _BATCH), jnp.int32),
            pltpu.SemaphoreType.DMA,
            pltpu.SemaphoreType.DMA,
        ],
    )(users, items, utT, itT)

    return pl.kernel(
        _dot_body,
        mesh=mesh,
        compiler_params=params,
        out_type=jax.ShapeDtypeStruct((_B,), jnp.float32),
        scratch_types=[
            pltpu.VMEM((_B_PER_W // 2, _SW), jnp.float32),
            pltpu.VMEM((_B_PER_W // 2, _SW), jnp.float32),
            pltpu.VMEM((_B_PER_W,), jnp.float32),
            pltpu.SemaphoreType.DMA,
            pltpu.SemaphoreType.DMA,
        ],
    )(ustage, istage)


def kernel(users, items, user_table, item_table):
    return _pure_mf_sc(users, items, user_table.T, item_table.T)
